# baseline (speedup 1.0000x reference)
import jax
import jax.numpy as jnp
from jax.experimental import pallas as pl
from jax.experimental.pallas import tpu as pltpu

NUM_K = 512
DIM = 512
T_LEN = 576
BATCH = 8
BB = 4
COMMIT = 0.25


def _vq_kernel(z_ref, w_ref, q_ref, idx_ref, loss_ref, acc_ref, sw_ref, wb_ref):
    b = pl.program_id(0)
    w = w_ref[...]

    @pl.when(b == 0)
    def _():
        sw_ref[...] = jnp.sum(w * w, axis=1, keepdims=True)
        wb_ref[...] = w.astype(jnp.bfloat16)

    ss = [jax.lax.dot_general(
              w, z_ref[j], (((1,), (0,)), ((), ())),
              preferred_element_type=jnp.float32) for j in range(BB)]

    iota_k = jax.lax.broadcasted_iota(jnp.int32, (NUM_K, T_LEN), 0)
    iota16 = jax.lax.broadcasted_iota(jnp.int16, (NUM_K, T_LEN), 0)
    step_sum = jnp.float32(0.0)
    for j in range(BB):
        zb = z_ref[j]
        sz = jnp.sum(zb * zb, axis=0)
        d = (sz[None, :] + sw_ref[...]) - 2.0 * ss[j]

        m = jnp.min(d, axis=0)
        idx = jnp.min(jnp.where(d == m[None, :], iota_k, NUM_K), axis=0)
        idx_ref[b * BB + j, :] = idx
        step_sum = step_sum + jnp.sum(m)

        enc = jnp.where(iota16 == idx[None, :].astype(jnp.int16),
                        jnp.bfloat16(1.0), jnp.bfloat16(0.0))
        q_ref[j] = jax.lax.dot_general(
            wb_ref[...], enc, (((0,), (0,)), ((), ())),
            preferred_element_type=jnp.float32)

    total = jnp.where(b == 0, 0.0, acc_ref[0]) + step_sum
    acc_ref[0] = total

    @pl.when(b == BATCH // BB - 1)
    def _():
        lat = total / jnp.float32(BATCH * DIM * T_LEN)
        loss_ref[0] = lat + COMMIT * lat


def kernel(z, embeddings_weight):
    q, idx, loss1 = pl.pallas_call(
        _vq_kernel,
        grid=(BATCH // BB,),
        in_specs=[
            pl.BlockSpec((BB, DIM, T_LEN), lambda b: (b, 0, 0)),
            pl.BlockSpec((NUM_K, DIM), lambda b: (0, 0)),
        ],
        out_specs=[
            pl.BlockSpec((BB, DIM, T_LEN), lambda b: (b, 0, 0)),
            pl.BlockSpec((BATCH, T_LEN), lambda b: (0, 0)),
            pl.BlockSpec(memory_space=pltpu.SMEM),
        ],
        out_shape=[
            jax.ShapeDtypeStruct((BATCH, DIM, T_LEN), jnp.float32),
            jax.ShapeDtypeStruct((BATCH, T_LEN), jnp.int32),
            jax.ShapeDtypeStruct((1,), jnp.float32),
        ],
        scratch_shapes=[
            pltpu.SMEM((1,), jnp.float32),
            pltpu.VMEM((NUM_K, 1), jnp.float32),
            pltpu.VMEM((NUM_K, DIM), jnp.bfloat16),
        ],
        compiler_params=pltpu.CompilerParams(
            dimension_semantics=("arbitrary",),
        ),
    )(z, embeddings_weight)
    return q, loss1[0], idx


# speedup vs baseline: 1.0293x; 1.0293x over previous
import jax
import jax.numpy as jnp
from jax.experimental import pallas as pl
from jax.experimental.pallas import tpu as pltpu

NUM_K = 512
DIM = 512
T_LEN = 576
BATCH = 8
BB = 2
COMMIT = 0.25


def _vq_kernel(z_ref, w_ref, q_ref, idx_ref, loss_ref, acc_ref, sw_ref, wb_ref):
    b = pl.program_id(0)
    w = w_ref[...]

    @pl.when(b == 0)
    def _():
        sw_ref[...] = jnp.sum(w * w, axis=1, keepdims=True)
        wb_ref[...] = w.astype(jnp.bfloat16)

    szs = [jnp.sum(z_ref[j] * z_ref[j], axis=0) for j in range(BB)]
    ss = [jax.lax.dot_general(
              w, z_ref[j], (((1,), (0,)), ((), ())),
              preferred_element_type=jnp.float32) for j in range(BB)]

    iota_k = jax.lax.broadcasted_iota(jnp.int32, (NUM_K, T_LEN), 0)
    iota16 = jax.lax.broadcasted_iota(jnp.int16, (NUM_K, T_LEN), 0)
    step_sum = jnp.float32(0.0)
    for j in range(BB):
        sz = szs[j]
        d = (sz[None, :] + sw_ref[...]) - 2.0 * ss[j]

        m = jnp.min(d, axis=0)
        idx = jnp.min(jnp.where(d == m[None, :], iota_k, NUM_K), axis=0)
        idx_ref[b * BB + j, :] = idx
        step_sum = step_sum + jnp.sum(m)

        enc = jnp.where(iota16 == idx[None, :].astype(jnp.int16),
                        jnp.bfloat16(1.0), jnp.bfloat16(0.0))
        q_ref[j] = jax.lax.dot_general(
            wb_ref[...], enc, (((0,), (0,)), ((), ())),
            preferred_element_type=jnp.float32)

    total = jnp.where(b == 0, 0.0, acc_ref[0]) + step_sum
    acc_ref[0] = total

    @pl.when(b == BATCH // BB - 1)
    def _():
        lat = total / jnp.float32(BATCH * DIM * T_LEN)
        loss_ref[0] = lat + COMMIT * lat


def kernel(z, embeddings_weight):
    q, idx, loss1 = pl.pallas_call(
        _vq_kernel,
        grid=(BATCH // BB,),
        in_specs=[
            pl.BlockSpec((BB, DIM, T_LEN), lambda b: (b, 0, 0)),
            pl.BlockSpec((NUM_K, DIM), lambda b: (0, 0)),
        ],
        out_specs=[
            pl.BlockSpec((BB, DIM, T_LEN), lambda b: (b, 0, 0)),
            pl.BlockSpec((BATCH, T_LEN), lambda b: (0, 0)),
            pl.BlockSpec(memory_space=pltpu.SMEM),
        ],
        out_shape=[
            jax.ShapeDtypeStruct((BATCH, DIM, T_LEN), jnp.float32),
            jax.ShapeDtypeStruct((BATCH, T_LEN), jnp.int32),
            jax.ShapeDtypeStruct((1,), jnp.float32),
        ],
        scratch_shapes=[
            pltpu.SMEM((1,), jnp.float32),
            pltpu.VMEM((NUM_K, 1), jnp.float32),
            pltpu.VMEM((NUM_K, DIM), jnp.bfloat16),
        ],
        compiler_params=pltpu.CompilerParams(
            dimension_semantics=("arbitrary",),
        ),
    )(z, embeddings_weight)
    return q, loss1[0], idx
